# baseline (speedup 1.0000x reference)
import functools

import jax
import jax.numpy as jnp
from jax import lax
from jax.experimental import pallas as pl
from jax.experimental.pallas import tpu as pltpu
from jax.experimental.pallas import tpu_sc as plsc

N_NODES = 10000
D = 128
B = 320000

NC = 2
NS = 16
L = 16

F_PER = D // NS
P_PER = F_PER // 2
B_PER_CORE = B // NC
E = 3200
N_CHUNK = B_PER_CORE // E
EG = E // L
SLICE = E // NS
RED_FULL = SLICE // L
GRP_UNROLL = 8


def _sc_body(zt_hbm, src_hbm, dst_hbm, out_hbm,
             zslice_v, src_v, dst_v, partial_v, red_v, res_v, stage_sh,
             idx_sem0, idx_sem1, stage_sem, read_sem):
    c = lax.axis_index("c")
    s = lax.axis_index("s")

    pltpu.sync_copy(zt_hbm.at[pl.ds(s * P_PER * N_NODES, P_PER * N_NODES)],
                    zslice_v)

    core_base = c * B_PER_CORE
    idx_sems = (idx_sem0, idx_sem1)

    def issue_idx(k, p, sem):
        kc = jnp.minimum(k, N_CHUNK - 1)
        off = core_base + kc * E
        pltpu.async_copy(src_hbm.at[pl.ds(off, E)],
                         src_v.at[pl.ds(p * E, E)], sem)
        pltpu.async_copy(dst_hbm.at[pl.ds(off, E)],
                         dst_v.at[pl.ds(p * E, E)], sem)

    def wait_idx(k, p, sem):
        off = core_base + k * E
        pltpu.make_async_copy(src_hbm.at[pl.ds(off, E)],
                              src_v.at[pl.ds(p * E, E)], sem).wait()
        pltpu.make_async_copy(dst_hbm.at[pl.ds(off, E)],
                              dst_v.at[pl.ds(p * E, E)], sem).wait()

    def compute(p):
        ibase = p * E

        @plsc.parallel_loop(0, EG, 1, unroll=GRP_UNROLL)
        def grp(g):
            sv = src_v[pl.ds(ibase + g * L, L)]
            dv = dst_v[pl.ds(ibase + g * L, L)]
            acc = jnp.zeros((L,), jnp.float32)
            for j in range(P_PER):
                wa = plsc.load_gather(zslice_v, [sv + (j * N_NODES)])
                wb = plsc.load_gather(zslice_v, [dv + (j * N_NODES)])
                a2 = plsc.bitcast(wa, jnp.bfloat16)
                b2 = plsc.bitcast(wb, jnp.bfloat16)
                p2 = a2 * b2
                pe, po = plsc.unpack(p2, format=plsc.PackFormat.INTERLEAVED)
                acc = acc + pe + po
            partial_v[pl.ds(g * L, L)] = acc

    def issue_stage_writes(s_, p):
        rbase = p * NS * E
        return [
            pltpu.async_copy(
                partial_v.at[pl.ds(t * SLICE, SLICE)],
                stage_sh.at[pl.ds(rbase + t * E + s_ * SLICE, SLICE)],
                stage_sem)
            for t in range(NS)
        ]

    def issue_red_read(s_, p):
        rbase = p * NS * E
        return pltpu.async_copy(
            stage_sh.at[pl.ds(rbase + s_ * E, E)], red_v, read_sem)

    def reduce_emit(k_prev, s_, read_h, out_sem):
        read_h.wait()

        def red_one(base):
            tot = jnp.zeros((L,), jnp.float32)
            for t in range(NS):
                tot = tot + red_v[pl.ds(t * SLICE + base, L)]
            y = 1.0 / (1.0 + jnp.exp(-tot))
            res_v[pl.ds(base, L)] = y

        @plsc.parallel_loop(0, RED_FULL, 1, unroll=2)
        def red(g):
            red_one(g * L)
        if SLICE % L != 0:
            red_one(SLICE - L)
        off_prev = core_base + k_prev * E
        return pltpu.async_copy(
            res_v, out_hbm.at[pl.ds(off_prev + s_ * SLICE, SLICE)], out_sem)

    def do_chunk(k, p, first=False):
        wait_idx(k, p, idx_sems[p])
        issue_idx(k + 1, 1 - p, idx_sems[1 - p])
        read_h = None if first else issue_red_read(s, 1 - p)
        compute(p)
        write_hs = issue_stage_writes(s, p)
        out_h = None
        if read_h is not None:
            out_h = reduce_emit(k - 1, s, read_h, read_sem)
        for h in write_hs:
            h.wait()
        if out_h is not None:
            out_h.wait()
        plsc.subcore_barrier()

    issue_idx(0, 0, idx_sems[0])

    def pair_body(i, carry):
        do_chunk(2 * i + 1, 1)
        do_chunk(2 * i + 2, 0)
        return carry

    do_chunk(0, 0, first=True)
    lax.fori_loop(0, (N_CHUNK - 2) // 2, pair_body, 0)
    do_chunk(N_CHUNK - 1, 1)
    wait_idx(N_CHUNK - 1, 0, idx_sems[0])
    read_h = issue_red_read(s, 1)
    out_h = reduce_emit(N_CHUNK - 1, s, read_h, read_sem)
    out_h.wait()


@jax.jit
def _predict(zt, src, dst):
    mesh = plsc.VectorSubcoreMesh(core_axis_name="c", subcore_axis_name="s")
    return pl.kernel(
        _sc_body,
        out_type=jax.ShapeDtypeStruct((B,), jnp.float32),
        mesh=mesh,
        compiler_params=pltpu.CompilerParams(needs_layout_passes=False),
        scratch_types=[
            pltpu.VMEM((P_PER * N_NODES,), jnp.int32),
            pltpu.VMEM((2 * E,), jnp.int32),
            pltpu.VMEM((2 * E,), jnp.int32),
            pltpu.VMEM((E,), jnp.float32),
            pltpu.VMEM((NS * SLICE,), jnp.float32),
            pltpu.VMEM((SLICE,), jnp.float32),
            pltpu.VMEM_SHARED((2 * NS * E,), jnp.float32),
            pltpu.SemaphoreType.DMA,
            pltpu.SemaphoreType.DMA,
            pltpu.SemaphoreType.DMA,
            pltpu.SemaphoreType.DMA,
        ],
    )(zt, src, dst)


def kernel(z, edge_index):
    zb = z.astype(jnp.bfloat16).T
    zp = zb.reshape(D // 2, 2, N_NODES)
    w = jnp.stack([zp[:, 0, :], zp[:, 1, :]], axis=-1)
    zt = jax.lax.bitcast_convert_type(w, jnp.int32).reshape(-1)
    src = edge_index[0].astype(jnp.int32)
    dst = edge_index[1].astype(jnp.int32)
    return _predict(zt, src, dst)
